# baseline (speedup 1.0000x reference)
import functools

import jax
import jax.numpy as jnp
from jax import lax
from jax.experimental import pallas as pl
from jax.experimental.pallas import tpu as pltpu
from jax.experimental.pallas import tpu_sc as plsc

V = 100000
D = 256
C = 128
B = 4096
LX = 200
LY = 50
LYP = 56

NC = 2
NS = 16
NW = NC * NS
EPW = B // NW
NSLOT = C // 16

MMB = 1000
NB = V // MMB
NRQ = (NB + 1) * MMB
BIAS_ROW = V
ZERO_ROW = V + 1


def _p_body(e_ref, wx_ref, p_ref):
    p_ref[...] = jnp.dot(e_ref[...].astype(jnp.bfloat16),
                         wx_ref[...].astype(jnp.bfloat16),
                         preferred_element_type=jnp.float32)


def _build_p(embed, wx):
    return pl.pallas_call(
        _p_body,
        grid=(NB,),
        in_specs=[
            pl.BlockSpec((MMB, D), lambda g: (g, 0)),
            pl.BlockSpec((D, C), lambda g: (0, 0)),
        ],
        out_specs=pl.BlockSpec((MMB, C), lambda g: (g, 0)),
        out_shape=jax.ShapeDtypeStruct((V, C), jnp.float32),
    )(embed, wx)


def _q_body(e_ref, wy_ref, b_ref, q_ref):
    g = pl.program_id(0)

    @pl.when(g < NB)
    def _():
        q_ref[...] = jnp.dot(e_ref[...].astype(jnp.bfloat16),
                             wy_ref[...].astype(jnp.bfloat16),
                             preferred_element_type=jnp.float32)

    @pl.when(g == NB)
    def _():
        q_ref[...] = jnp.zeros_like(q_ref)
        q_ref[0:1, :] = b_ref[...]


def _build_q(embed, wy, bias2d):
    return pl.pallas_call(
        _q_body,
        grid=(NB + 1,),
        in_specs=[
            pl.BlockSpec((MMB, D), lambda g: (g % NB, 0)),
            pl.BlockSpec((D, C), lambda g: (0, 0)),
            pl.BlockSpec((1, C), lambda g: (0, 0)),
        ],
        out_specs=pl.BlockSpec((MMB, C), lambda g: (g, 0)),
        out_shape=jax.ShapeDtypeStruct((NRQ, C), jnp.float32),
    )(embed, wy, bias2d)


def _sum_rows(rows_ref, accs, n, unroll):
    def body(r0, accs):
        for u in range(unroll):
            accs = tuple(
                accs[j] + rows_ref[r0 * unroll + u, pl.ds(16 * j, 16)]
                for j in range(NSLOT)
            )
        return accs
    return lax.fori_loop(0, n // unroll, body, accs)


def _make_pool(L, init_from_partial):
    scratch = [
        pltpu.VMEM((EPW * L,), jnp.int32),
        pltpu.VMEM((L, C), jnp.float32),
        pltpu.VMEM((L, C), jnp.float32),
        pltpu.VMEM((EPW * C,), jnp.float32),
        pltpu.SemaphoreType.DMA,
        pltpu.SemaphoreType.DMA,
    ]

    def body(tbl_hbm, if_hbm, *rest):
        if init_from_partial:
            part_hbm, out_hbm, idxv, r0, r1, outv, s0, s1 = rest
        else:
            out_hbm, idxv, r0, r1, outv, s0, s1 = rest
        wid = lax.axis_index("s") * NC + lax.axis_index("c")
        pltpu.sync_copy(if_hbm.at[pl.ds(wid * (EPW * L), EPW * L)], idxv)
        if init_from_partial:
            pltpu.sync_copy(part_hbm.at[pl.ds(wid * (EPW * C), EPW * C)], outv)

        bufs = ((r0, s0), (r1, s1))

        def issue(e, buf):
            rb, sb = buf
            pltpu.async_copy(tbl_hbm.at[idxv.at[pl.ds(e * L, L)]], rb, sb)

        def drain_and_sum(e, buf):
            rb, sb = buf
            if init_from_partial:
                init = tuple(outv[pl.ds(e * C + 16 * j, 16)]
                             for j in range(NSLOT))
            else:
                init = tuple(jnp.zeros((16,), jnp.float32)
                             for _ in range(NSLOT))
            pltpu.make_async_copy(tbl_hbm.at[pl.ds(0, L)], rb, sb).wait()
            accs = _sum_rows(rb, init, L, 4)
            for j in range(NSLOT):
                outv[pl.ds(e * C + 16 * j, 16)] = accs[j]

        issue(0, bufs[0])

        @pl.loop(0, EPW, step=2)
        def _(e0):
            for p in range(2):
                e = e0 + p

                @pl.when(e + 1 < EPW)
                def _():
                    issue(e + 1, bufs[(p + 1) % 2])

                drain_and_sum(e, bufs[p])

        pltpu.sync_copy(outv, out_hbm.at[pl.ds(wid * (EPW * C), EPW * C)])

    return functools.partial(
        pl.kernel,
        mesh=plsc.VectorSubcoreMesh(core_axis_name="c", subcore_axis_name="s"),
        out_type=jax.ShapeDtypeStruct((B * C,), jnp.float32),
        scratch_types=scratch,
    )(body)


_pool_x = _make_pool(LX, init_from_partial=False)
_pool_y = _make_pool(LYP, init_from_partial=True)


def kernel(x, y, embed, W, b):
    wx = jnp.transpose(W[:, :D]) / LX
    wy = jnp.transpose(W[:, D:]) / LY
    bias2d = b.reshape(1, C).astype(jnp.float32)
    p_tbl = _build_p(embed, wx)
    q_tbl = _build_q(embed, wy, bias2d)

    xf = x.astype(jnp.int32).reshape(-1)
    pad = jnp.full((B, LYP - LY), ZERO_ROW, jnp.int32).at[:, 0].set(BIAS_ROW)
    yf = jnp.concatenate([y.astype(jnp.int32), pad], axis=1).reshape(-1)

    part = _pool_x(p_tbl, xf)
    out = _pool_y(q_tbl, yf, part)
    return out.reshape(B, C)


# speedup vs baseline: 1.0132x; 1.0132x over previous
import functools

import jax
import jax.numpy as jnp
from jax import lax
from jax.experimental import pallas as pl
from jax.experimental.pallas import tpu as pltpu
from jax.experimental.pallas import tpu_sc as plsc

V = 100000
D = 256
C = 128
B = 4096
LX = 200
LY = 50
LYP = 56

NC = 2
NS = 16
NW = NC * NS
EPW = B // NW
NSLOT = C // 16

XA = 128
XB = LX - XA

MMB = 1000
NB = V // MMB
NR = (NB + 1) * MMB
BIAS_ROW = V
ZERO_ROW = V + 1


def _tbl_body(e_ref, wx_ref, wy_ref, b_ref, p_ref, q_ref):
    g = pl.program_id(0)

    @pl.when(g < NB)
    def _():
        eb = e_ref[...].astype(jnp.bfloat16)
        p_ref[...] = jnp.dot(eb, wx_ref[...].astype(jnp.bfloat16),
                             preferred_element_type=jnp.float32)
        q_ref[...] = jnp.dot(eb, wy_ref[...].astype(jnp.bfloat16),
                             preferred_element_type=jnp.float32)

    @pl.when(g == NB)
    def _():
        p_ref[...] = jnp.zeros_like(p_ref)
        q_ref[...] = jnp.zeros_like(q_ref)
        q_ref[0:1, :] = b_ref[...]


def _build_tables(embed, wx, wy, bias2d):
    return pl.pallas_call(
        _tbl_body,
        grid=(NB + 1,),
        in_specs=[
            pl.BlockSpec((MMB, D), lambda g: (g % NB, 0)),
            pl.BlockSpec((D, C), lambda g: (0, 0)),
            pl.BlockSpec((D, C), lambda g: (0, 0)),
            pl.BlockSpec((1, C), lambda g: (0, 0)),
        ],
        out_specs=[
            pl.BlockSpec((MMB, C), lambda g: (g, 0)),
            pl.BlockSpec((MMB, C), lambda g: (g, 0)),
        ],
        out_shape=[
            jax.ShapeDtypeStruct((NR, C), jnp.float32),
            jax.ShapeDtypeStruct((NR, C), jnp.float32),
        ],
    )(embed, wx, wy, bias2d)


def _sum_rows(rows_ref, accs, n, unroll):
    def body(r0, accs):
        for u in range(unroll):
            accs = tuple(
                accs[j] + rows_ref[r0 * unroll + u, pl.ds(16 * j, 16)]
                for j in range(NSLOT)
            )
        return accs
    return lax.fori_loop(0, n // unroll, body, accs)


@functools.partial(
    pl.kernel,
    mesh=plsc.VectorSubcoreMesh(core_axis_name="c", subcore_axis_name="s"),
    out_type=jax.ShapeDtypeStruct((B * C,), jnp.float32),
    scratch_types=[
        pltpu.VMEM((EPW * LX,), jnp.int32),
        pltpu.VMEM((EPW * LYP,), jnp.int32),
        pltpu.VMEM((LX, C), jnp.float32),
        pltpu.VMEM((LYP, C), jnp.float32),
        pltpu.VMEM((LX, C), jnp.float32),
        pltpu.VMEM((LYP, C), jnp.float32),
        pltpu.VMEM((EPW * C,), jnp.float32),
        pltpu.SemaphoreType.DMA,
        pltpu.SemaphoreType.DMA,
        pltpu.SemaphoreType.DMA,
        pltpu.SemaphoreType.DMA,
    ],
)
def _sc_pool(p_hbm, q_hbm, xf_hbm, yf_hbm, out_hbm,
             idxx, idxy, a0, c0, a1, c1, outv,
             sa0, sc0, sa1, sc1):
    wid = lax.axis_index("s") * NC + lax.axis_index("c")
    pltpu.sync_copy(xf_hbm.at[pl.ds(wid * (EPW * LX), EPW * LX)], idxx)
    pltpu.sync_copy(yf_hbm.at[pl.ds(wid * (EPW * LYP), EPW * LYP)], idxy)

    bufs = ((a0, c0, sa0, sc0), (a1, c1, sa1, sc1))

    def issue(e, buf):
        ra, rc, sa, sc = buf
        pltpu.async_copy(p_hbm.at[idxx.at[pl.ds(e * LX, LX)]], ra, sa)
        pltpu.async_copy(q_hbm.at[idxy.at[pl.ds(e * LYP, LYP)]], rc, sc)

    def drain_and_sum(e, buf):
        ra, rc, sa, sc = buf
        zeros = tuple(jnp.zeros((16,), jnp.float32) for _ in range(NSLOT))
        pltpu.make_async_copy(p_hbm.at[pl.ds(0, LX)], ra, sa).wait()
        accs = _sum_rows(ra, zeros, LX, 4)
        pltpu.make_async_copy(q_hbm.at[pl.ds(0, LYP)], rc, sc).wait()
        accs = _sum_rows(rc, accs, LYP, 4)
        for j in range(NSLOT):
            outv[pl.ds(e * C + 16 * j, 16)] = accs[j]

    issue(0, bufs[0])

    @pl.loop(0, EPW, step=2)
    def _(e0):
        for p in range(2):
            e = e0 + p

            @pl.when(e + 1 < EPW)
            def _():
                issue(e + 1, bufs[(p + 1) % 2])

            drain_and_sum(e, bufs[p])

    pltpu.sync_copy(outv, out_hbm.at[pl.ds(wid * (EPW * C), EPW * C)])


def kernel(x, y, embed, W, b):
    wx = jnp.transpose(W[:, :D]) / LX
    wy = jnp.transpose(W[:, D:]) / LY
    bias2d = b.reshape(1, C).astype(jnp.float32)
    p_tbl, q_tbl = _build_tables(embed, wx, wy, bias2d)

    xf = x.astype(jnp.int32).reshape(-1)
    pad = jnp.full((B, LYP - LY), ZERO_ROW, jnp.int32).at[:, 0].set(BIAS_ROW)
    yf = jnp.concatenate([y.astype(jnp.int32), pad], axis=1).reshape(-1)

    out = _sc_pool(p_tbl, q_tbl, xf, yf)
    return out.reshape(B, C)


# speedup vs baseline: 1.0203x; 1.0070x over previous
import functools

import jax
import jax.numpy as jnp
from jax import lax
from jax.experimental import pallas as pl
from jax.experimental.pallas import tpu as pltpu
from jax.experimental.pallas import tpu_sc as plsc

V = 100000
D = 256
C = 128
B = 4096
LX = 200
LY = 50
LYP = 56

NC = 2
NS = 16
NW = NC * NS
EPW = B // NW
NSLOT = C // 16

XA = 128
XB = LX - XA

MMB = 2000
NB = V // MMB
NR = (NB + 1) * MMB
BIAS_ROW = V
ZERO_ROW = V + 1


def _tbl_body(e_ref, wx_ref, wy_ref, b_ref, p_ref, q_ref):
    g = pl.program_id(0)

    @pl.when(g < NB)
    def _():
        eb = e_ref[...].astype(jnp.bfloat16)
        p_ref[...] = jnp.dot(eb, wx_ref[...].astype(jnp.bfloat16),
                             preferred_element_type=jnp.float32)
        q_ref[...] = jnp.dot(eb, wy_ref[...].astype(jnp.bfloat16),
                             preferred_element_type=jnp.float32)

    @pl.when(g == NB)
    def _():
        p_ref[...] = jnp.zeros_like(p_ref)
        q_ref[...] = jnp.zeros_like(q_ref)
        q_ref[0:1, :] = b_ref[...]


def _build_tables(embed, wx, wy, bias2d):
    return pl.pallas_call(
        _tbl_body,
        grid=(NB + 1,),
        in_specs=[
            pl.BlockSpec((MMB, D), lambda g: (g % NB, 0)),
            pl.BlockSpec((D, C), lambda g: (0, 0)),
            pl.BlockSpec((D, C), lambda g: (0, 0)),
            pl.BlockSpec((1, C), lambda g: (0, 0)),
        ],
        out_specs=[
            pl.BlockSpec((MMB, C), lambda g: (g, 0)),
            pl.BlockSpec((MMB, C), lambda g: (g, 0)),
        ],
        out_shape=[
            jax.ShapeDtypeStruct((NR, C), jnp.float32),
            jax.ShapeDtypeStruct((NR, C), jnp.float32),
        ],
    )(embed, wx, wy, bias2d)


def _sum_rows(rows_ref, accs, n, unroll):
    def body(r0, accs):
        for u in range(unroll):
            accs = tuple(
                accs[j] + rows_ref[r0 * unroll + u, pl.ds(16 * j, 16)]
                for j in range(NSLOT)
            )
        return accs
    return lax.fori_loop(0, n // unroll, body, accs)


@functools.partial(
    pl.kernel,
    mesh=plsc.VectorSubcoreMesh(core_axis_name="c", subcore_axis_name="s"),
    out_type=jax.ShapeDtypeStruct((B * C,), jnp.float32),
    scratch_types=[
        pltpu.VMEM((EPW * LX,), jnp.int32),
        pltpu.VMEM((EPW * LYP,), jnp.int32),
        pltpu.VMEM((LX, C), jnp.float32),
        pltpu.VMEM((LYP, C), jnp.float32),
        pltpu.VMEM((LX, C), jnp.float32),
        pltpu.VMEM((LYP, C), jnp.float32),
        pltpu.VMEM((EPW * C,), jnp.float32),
        pltpu.SemaphoreType.DMA,
        pltpu.SemaphoreType.DMA,
        pltpu.SemaphoreType.DMA,
        pltpu.SemaphoreType.DMA,
    ],
)
def _sc_pool(p_hbm, q_hbm, xf_hbm, yf_hbm, out_hbm,
             idxx, idxy, a0, c0, a1, c1, outv,
             sa0, sc0, sa1, sc1):
    wid = lax.axis_index("s") * NC + lax.axis_index("c")
    pltpu.sync_copy(xf_hbm.at[pl.ds(wid * (EPW * LX), EPW * LX)], idxx)
    pltpu.sync_copy(yf_hbm.at[pl.ds(wid * (EPW * LYP), EPW * LYP)], idxy)

    bufs = ((a0, c0, sa0, sc0), (a1, c1, sa1, sc1))

    def issue(e, buf):
        ra, rc, sa, sc = buf
        pltpu.async_copy(p_hbm.at[idxx.at[pl.ds(e * LX, LX)]], ra, sa)
        pltpu.async_copy(q_hbm.at[idxy.at[pl.ds(e * LYP, LYP)]], rc, sc)

    def drain_and_sum(e, buf):
        ra, rc, sa, sc = buf
        zeros = tuple(jnp.zeros((16,), jnp.float32) for _ in range(NSLOT))
        pltpu.make_async_copy(p_hbm.at[pl.ds(0, LX)], ra, sa).wait()
        accs = _sum_rows(ra, zeros, LX, 4)
        pltpu.make_async_copy(q_hbm.at[pl.ds(0, LYP)], rc, sc).wait()
        accs = _sum_rows(rc, accs, LYP, 4)
        for j in range(NSLOT):
            outv[pl.ds(e * C + 16 * j, 16)] = accs[j]

    issue(0, bufs[0])

    @pl.loop(0, EPW, step=2)
    def _(e0):
        for p in range(2):
            e = e0 + p

            @pl.when(e + 1 < EPW)
            def _():
                issue(e + 1, bufs[(p + 1) % 2])

            drain_and_sum(e, bufs[p])

    pltpu.sync_copy(outv, out_hbm.at[pl.ds(wid * (EPW * C), EPW * C)])


def kernel(x, y, embed, W, b):
    wx = jnp.transpose(W[:, :D]) / LX
    wy = jnp.transpose(W[:, D:]) / LY
    bias2d = b.reshape(1, C).astype(jnp.float32)
    p_tbl, q_tbl = _build_tables(embed, wx, wy, bias2d)

    xf = x.astype(jnp.int32).reshape(-1)
    pad = jnp.full((B, LYP - LY), ZERO_ROW, jnp.int32).at[:, 0].set(BIAS_ROW)
    yf = jnp.concatenate([y.astype(jnp.int32), pad], axis=1).reshape(-1)

    out = _sc_pool(p_tbl, q_tbl, xf, yf)
    return out.reshape(B, C)


# speedup vs baseline: 1.0503x; 1.0294x over previous
import functools

import jax
import jax.numpy as jnp
from jax import lax
from jax.experimental import pallas as pl
from jax.experimental.pallas import tpu as pltpu
from jax.experimental.pallas import tpu_sc as plsc

V = 100000
D = 256
C = 128
B = 4096
LX = 200
LY = 50
LYP = 56

NC = 2
NS = 16
NW = NC * NS
EPW = B // NW
NSLOT = C // 16

XA = 128
XB = LX - XA

MMB = 4000
NB = V // MMB
NR = (NB + 1) * MMB
BIAS_ROW = V
ZERO_ROW = V + 1


def _tbl_body(e_ref, wx_ref, wy_ref, b_ref, p_ref, q_ref):
    g = pl.program_id(0)

    @pl.when(g < NB)
    def _():
        eb = e_ref[...].astype(jnp.bfloat16)
        p_ref[...] = jnp.dot(eb, wx_ref[...].astype(jnp.bfloat16),
                             preferred_element_type=jnp.float32)
        q_ref[...] = jnp.dot(eb, wy_ref[...].astype(jnp.bfloat16),
                             preferred_element_type=jnp.float32)

    @pl.when(g == NB)
    def _():
        p_ref[...] = jnp.zeros_like(p_ref)
        q_ref[...] = jnp.zeros_like(q_ref)
        q_ref[0:1, :] = b_ref[...]


def _build_tables(embed, wx, wy, bias2d):
    return pl.pallas_call(
        _tbl_body,
        grid=(NB + 1,),
        in_specs=[
            pl.BlockSpec((MMB, D), lambda g: (g % NB, 0)),
            pl.BlockSpec((D, C), lambda g: (0, 0)),
            pl.BlockSpec((D, C), lambda g: (0, 0)),
            pl.BlockSpec((1, C), lambda g: (0, 0)),
        ],
        out_specs=[
            pl.BlockSpec((MMB, C), lambda g: (g, 0)),
            pl.BlockSpec((MMB, C), lambda g: (g, 0)),
        ],
        out_shape=[
            jax.ShapeDtypeStruct((NR, C), jnp.float32),
            jax.ShapeDtypeStruct((NR, C), jnp.float32),
        ],
    )(embed, wx, wy, bias2d)


def _sum_rows(rows_ref, accs, n, unroll):
    def body(r0, accs):
        for u in range(unroll):
            accs = tuple(
                accs[j] + rows_ref[r0 * unroll + u, pl.ds(16 * j, 16)]
                for j in range(NSLOT)
            )
        return accs
    return lax.fori_loop(0, n // unroll, body, accs)


@functools.partial(
    pl.kernel,
    mesh=plsc.VectorSubcoreMesh(core_axis_name="c", subcore_axis_name="s"),
    out_type=jax.ShapeDtypeStruct((B * C,), jnp.float32),
    scratch_types=[
        pltpu.VMEM((EPW * LX,), jnp.int32),
        pltpu.VMEM((EPW * LYP,), jnp.int32),
        pltpu.VMEM((LX, C), jnp.float32),
        pltpu.VMEM((LYP, C), jnp.float32),
        pltpu.VMEM((LX, C), jnp.float32),
        pltpu.VMEM((LYP, C), jnp.float32),
        pltpu.VMEM((EPW * C,), jnp.float32),
        pltpu.SemaphoreType.DMA,
        pltpu.SemaphoreType.DMA,
        pltpu.SemaphoreType.DMA,
        pltpu.SemaphoreType.DMA,
    ],
)
def _sc_pool(p_hbm, q_hbm, xf_hbm, yf_hbm, out_hbm,
             idxx, idxy, a0, c0, a1, c1, outv,
             sa0, sc0, sa1, sc1):
    wid = lax.axis_index("s") * NC + lax.axis_index("c")
    pltpu.sync_copy(xf_hbm.at[pl.ds(wid * (EPW * LX), EPW * LX)], idxx)
    pltpu.sync_copy(yf_hbm.at[pl.ds(wid * (EPW * LYP), EPW * LYP)], idxy)

    bufs = ((a0, c0, sa0, sc0), (a1, c1, sa1, sc1))

    def issue(e, buf):
        ra, rc, sa, sc = buf
        pltpu.async_copy(p_hbm.at[idxx.at[pl.ds(e * LX, LX)]], ra, sa)
        pltpu.async_copy(q_hbm.at[idxy.at[pl.ds(e * LYP, LYP)]], rc, sc)

    def drain_and_sum(e, buf):
        ra, rc, sa, sc = buf
        zeros = tuple(jnp.zeros((16,), jnp.float32) for _ in range(NSLOT))
        pltpu.make_async_copy(p_hbm.at[pl.ds(0, LX)], ra, sa).wait()
        accs = _sum_rows(ra, zeros, LX, 4)
        pltpu.make_async_copy(q_hbm.at[pl.ds(0, LYP)], rc, sc).wait()
        accs = _sum_rows(rc, accs, LYP, 4)
        for j in range(NSLOT):
            outv[pl.ds(e * C + 16 * j, 16)] = accs[j]

    issue(0, bufs[0])

    @pl.loop(0, EPW, step=2)
    def _(e0):
        for p in range(2):
            e = e0 + p

            @pl.when(e + 1 < EPW)
            def _():
                issue(e + 1, bufs[(p + 1) % 2])

            drain_and_sum(e, bufs[p])

    pltpu.sync_copy(outv, out_hbm.at[pl.ds(wid * (EPW * C), EPW * C)])


def kernel(x, y, embed, W, b):
    wx = jnp.transpose(W[:, :D]) / LX
    wy = jnp.transpose(W[:, D:]) / LY
    bias2d = b.reshape(1, C).astype(jnp.float32)
    p_tbl, q_tbl = _build_tables(embed, wx, wy, bias2d)

    xf = x.astype(jnp.int32).reshape(-1)
    pad = jnp.full((B, LYP - LY), ZERO_ROW, jnp.int32).at[:, 0].set(BIAS_ROW)
    yf = jnp.concatenate([y.astype(jnp.int32), pad], axis=1).reshape(-1)

    out = _sc_pool(p_tbl, q_tbl, xf, yf)
    return out.reshape(B, C)
